# baseline (speedup 1.0000x reference)
import functools
import math

import jax
import jax.numpy as jnp
from jax import lax
from jax.experimental import pallas as pl
from jax.experimental.pallas import tpu as pltpu
from jax.experimental.pallas import tpu_sc as plsc

_D = 64
_N = 4096 * 200
_NC = 2
_NS = 16
_NW = _NC * _NS
_BPW = _N // _NW
_KG = 128
_K = 512
_G = _K // _KG
_NCH = _BPW // _K
_SCALE = 1.0 / math.sqrt(_D)

_mesh = plsc.VectorSubcoreMesh(core_axis_name="c", subcore_axis_name="s")


@functools.partial(
    pl.kernel,
    out_type=jax.ShapeDtypeStruct((_N, _D), jnp.float32),
    mesh=_mesh,
    scratch_types=[
        pltpu.VMEM((_G, _KG), jnp.int32),
        pltpu.VMEM((_K, _D), jnp.float32),
        pltpu.SemaphoreType.DMA,
    ],
    compiler_params=pltpu.CompilerParams(use_tc_tiling_on_sc=False),
)
def _emb_lookup(xr_hbm, table_hbm, out_hbm, idx_v, rows_v, sem):
    wid = lax.axis_index("s") * _NC + lax.axis_index("c")
    idx_row0 = wid * (_BPW // _KG)
    out_row0 = wid * _BPW

    @pl.loop(0, _NCH)
    def _chunk(c):
        pltpu.sync_copy(xr_hbm.at[pl.ds(idx_row0 + c * _G, _G)], idx_v)
        for g in range(_G):
            pltpu.async_copy(
                table_hbm.at[idx_v.at[g]],
                rows_v.at[pl.ds(g * _KG, _KG)],
                sem,
            ).wait()

        @pl.loop(0, _K)
        def _row(r):
            for j in range(_D // 16):
                sl = pl.ds(j * 16, 16)
                rows_v[r, sl] = rows_v[r, sl] * _SCALE

        pltpu.sync_copy(rows_v, out_hbm.at[pl.ds(out_row0 + c * _K, _K)])


def kernel(x, table):
    xr = x.reshape(_N // _KG, _KG)
    out = _emb_lookup(xr, table)
    return out.reshape(x.shape[0], x.shape[1], _D)


# speedup vs baseline: 1.2054x; 1.2054x over previous
import functools
import math

import jax
import jax.numpy as jnp
from jax import lax
from jax.experimental import pallas as pl
from jax.experimental.pallas import tpu as pltpu
from jax.experimental.pallas import tpu_sc as plsc

_D = 64
_N = 4096 * 200
_NC = 2
_NS = 16
_NW = _NC * _NS
_BPW = _N // _NW
_KG = 128
_K = 512
_G = _K // _KG
_NCH = _BPW // _K
_SCALE = 1.0 / math.sqrt(_D)

_mesh = plsc.VectorSubcoreMesh(core_axis_name="c", subcore_axis_name="s")


@functools.partial(
    pl.kernel,
    out_type=jax.ShapeDtypeStruct((_N, _D), jnp.float32),
    mesh=_mesh,
    scratch_types=[
        pltpu.VMEM((2, _G, _KG), jnp.int32),
        pltpu.VMEM((2, _K, _D), jnp.float32),
        pltpu.SemaphoreType.DMA,
        pltpu.SemaphoreType.DMA,
        pltpu.SemaphoreType.DMA,
        pltpu.SemaphoreType.DMA,
        pltpu.SemaphoreType.DMA,
        pltpu.SemaphoreType.DMA,
    ],
    compiler_params=pltpu.CompilerParams(use_tc_tiling_on_sc=False),
)
def _emb_lookup(xr_hbm, table_hbm, out_hbm, idx_v, rows_v,
                isem0, isem1, gsem0, gsem1, osem0, osem1):
    isem = (isem0, isem1)
    gsem = (gsem0, gsem1)
    osem = (osem0, osem1)
    wid = lax.axis_index("s") * _NC + lax.axis_index("c")
    idx_row0 = wid * (_BPW // _KG)
    out_row0 = wid * _BPW

    def load_idx(c, b):
        pltpu.async_copy(
            xr_hbm.at[pl.ds(idx_row0 + c * _G, _G)], idx_v.at[b], isem[b])

    def wait_idx(b):
        pltpu.make_async_copy(
            xr_hbm.at[pl.ds(idx_row0, _G)], idx_v.at[b], isem[b]).wait()

    def fire_gathers(b):
        for g in range(_G):
            pltpu.async_copy(
                table_hbm.at[idx_v.at[b, g]],
                rows_v.at[b, pl.ds(g * _KG, _KG)],
                gsem[b],
            )

    def wait_gathers(b):
        for g in range(_G):
            pltpu.make_async_copy(
                table_hbm.at[idx_v.at[b, g]],
                rows_v.at[b, pl.ds(g * _KG, _KG)],
                gsem[b],
            ).wait()

    def scale(b):
        @pl.loop(0, _K, unroll=4)
        def _row(r):
            for j in range(_D // 16):
                sl = pl.ds(j * 16, 16)
                rows_v[b, r, sl] = rows_v[b, r, sl] * _SCALE

    def fire_store(c, b):
        pltpu.async_copy(
            rows_v.at[b], out_hbm.at[pl.ds(out_row0 + c * _K, _K)], osem[b])

    def wait_store(b):
        pltpu.make_async_copy(
            rows_v.at[b], out_hbm.at[pl.ds(out_row0, _K)], osem[b]).wait()

    load_idx(0, 0)
    load_idx(1, 1)
    wait_idx(0)
    fire_gathers(0)

    for c in range(_NCH):
        b = c % 2
        nb = 1 - b
        wait_gathers(b)
        if c + 2 < _NCH:
            load_idx(c + 2, b)
        if c + 1 < _NCH:
            wait_idx(nb)
            if c >= 1:
                wait_store(nb)
            fire_gathers(nb)
        scale(b)
        fire_store(c, b)
    wait_store((_NCH - 1) % 2)


def kernel(x, table):
    xr = x.reshape(_N // _KG, _KG)
    out = _emb_lookup(xr, table)
    return out.reshape(x.shape[0], x.shape[1], _D)


# speedup vs baseline: 1.4647x; 1.2151x over previous
import functools
import math

import jax
import jax.numpy as jnp
from jax import lax
from jax.experimental import pallas as pl
from jax.experimental.pallas import tpu as pltpu
from jax.experimental.pallas import tpu_sc as plsc

_D = 64
_DP = 128
_N = 4096 * 200
_NC = 2
_NS = 16
_NW = _NC * _NS
_BPW = _N // _NW
_KG = 128
_K = 256
_G = _K // _KG
_NB = 3
_F = 1
_NCH = _BPW // _K
_SCALE = 1.0 / math.sqrt(_D)

_mesh = plsc.VectorSubcoreMesh(core_axis_name="c", subcore_axis_name="s")


@functools.partial(
    pl.kernel,
    out_type=jax.ShapeDtypeStruct((_N, _DP), jnp.float32),
    mesh=_mesh,
    scratch_types=[
        pltpu.VMEM((_NB, _G, _KG), jnp.int32),
        pltpu.VMEM((_NB, _K, _DP), jnp.float32),
    ] + [pltpu.SemaphoreType.DMA] * (3 * _NB),
)
def _emb_lookup(xr_hbm, table_hbm, out_hbm, idx_v, rows_v, *sems):
    isem = sems[0:_NB]
    gsem = sems[_NB:2 * _NB]
    osem = sems[2 * _NB:3 * _NB]
    wid = lax.axis_index("s") * _NC + lax.axis_index("c")
    idx_row0 = wid * (_BPW // _KG)
    out_row0 = wid * _BPW

    def load_idx(c, b):
        pltpu.async_copy(
            xr_hbm.at[pl.ds(idx_row0 + c * _G, _G)], idx_v.at[b], isem[b])

    def wait_idx(b):
        pltpu.make_async_copy(
            xr_hbm.at[pl.ds(idx_row0, _G)], idx_v.at[b], isem[b]).wait()

    def fire_gathers(b):
        for g in range(_G):
            pltpu.async_copy(
                table_hbm.at[idx_v.at[b, g]],
                rows_v.at[b, pl.ds(g * _KG, _KG)],
                gsem[b],
            )

    def wait_gathers(b):
        for g in range(_G):
            pltpu.make_async_copy(
                table_hbm.at[idx_v.at[b, g]],
                rows_v.at[b, pl.ds(g * _KG, _KG)],
                gsem[b],
            ).wait()

    def scale(b):
        @pl.loop(0, _K, unroll=4)
        def _row(r):
            for j in range(_D // 16):
                sl = pl.ds(j * 16, 16)
                rows_v[b, r, sl] = rows_v[b, r, sl] * _SCALE

    def fire_store(c, b):
        pltpu.async_copy(
            rows_v.at[b],
            out_hbm.at[pl.ds(out_row0 + c * _K, _K)], osem[b])

    def wait_store(b):
        pltpu.make_async_copy(
            rows_v.at[b],
            out_hbm.at[pl.ds(out_row0, _K)], osem[b]).wait()

    for k in range(_NB):
        load_idx(k, k)
    for k in range(_F):
        wait_idx(k)
        fire_gathers(k)

    for c in range(_NCH):
        s = c % _NB
        wait_gathers(s)
        if c + _NB < _NCH:
            load_idx(c + _NB, s)
        if c + _F < _NCH:
            t = (c + _F) % _NB
            wait_idx(t)
            if c + _F >= _NB:
                wait_store(t)
            fire_gathers(t)
        scale(s)
        fire_store(c, s)
    for k in range(_NCH - _NB, _NCH):
        wait_store(k % _NB)


def kernel(x, table):
    xr = x.reshape(_N // _KG, _KG)
    tp = jnp.pad(table, ((0, 0), (0, _DP - _D)))
    out = _emb_lookup(xr, tp)
    return out[:, :_D].reshape(x.shape[0], x.shape[1], _D)


# speedup vs baseline: 1.4720x; 1.0050x over previous
import functools
import math

import jax
import jax.numpy as jnp
from jax import lax
from jax.experimental import pallas as pl
from jax.experimental.pallas import tpu as pltpu
from jax.experimental.pallas import tpu_sc as plsc

_D = 64
_DP = 128
_N = 4096 * 200
_NC = 2
_NS = 16
_NW = _NC * _NS
_BPW = _N // _NW
_K = 128
_NB = 6
_F = 4
_NCH = _BPW // _K
_SCALE = 1.0 / math.sqrt(_D)

_C0 = _NB
_NSUP = (_NCH - _F - _C0) // _NB
_C1 = _C0 + _NSUP * _NB

_mesh = plsc.VectorSubcoreMesh(core_axis_name="c", subcore_axis_name="s")


@functools.partial(
    pl.kernel,
    out_type=jax.ShapeDtypeStruct((_N, _DP), jnp.float32),
    mesh=_mesh,
    scratch_types=[
        pltpu.VMEM((_NCH, _K), jnp.int32),
        pltpu.VMEM((_NB, _K, _DP), jnp.float32),
    ] + [pltpu.SemaphoreType.DMA] * (2 * _NB),
)
def _emb_lookup(xr_hbm, table_hbm, out_hbm, idx_v, rows_v, *sems):
    gsem = sems[0:_NB]
    osem = sems[_NB:2 * _NB]
    wid = lax.axis_index("s") * _NC + lax.axis_index("c")
    idx_row0 = wid * _NCH
    out_row0 = wid * _BPW

    def fire_gather(c, b):
        pltpu.async_copy(
            table_hbm.at[idx_v.at[c]], rows_v.at[b], gsem[b])

    def wait_gather(b):
        pltpu.make_async_copy(
            table_hbm.at[idx_v.at[0]], rows_v.at[b], gsem[b]).wait()

    def scale(b):
        @pl.loop(0, _K, unroll=4)
        def _row(r):
            for j in range(_D // 16):
                sl = pl.ds(j * 16, 16)
                rows_v[b, r, sl] = rows_v[b, r, sl] * _SCALE

    def fire_store(c, b):
        pltpu.async_copy(
            rows_v.at[b], out_hbm.at[pl.ds(out_row0 + c * _K, _K)], osem[b])

    def wait_store(b):
        pltpu.make_async_copy(
            rows_v.at[b], out_hbm.at[pl.ds(out_row0, _K)], osem[b]).wait()

    def step(c, s, guarded):
        wait_gather(s)
        t = (s + _F) % _NB
        if guarded:
            if c + _F < _NCH:
                if c + _F >= _NB:
                    wait_store(t)
                fire_gather(c + _F, t)
        else:
            wait_store(t)
            fire_gather(c + _F, t)
        scale(s)
        fire_store(c, s)

    pltpu.sync_copy(xr_hbm.at[pl.ds(idx_row0, _NCH)], idx_v)

    for k in range(_F):
        fire_gather(k, k)
    for c in range(_C0):
        step(c, c % _NB, True)

    @pl.loop(0, _NSUP)
    def _sup(sp):
        c0 = _C0 + sp * _NB
        for j in range(_NB):
            step(c0 + j, j, False)

    for c in range(_C1, _NCH):
        step(c, c % _NB, True)
    for k in range(_NCH - _NB, _NCH):
        wait_store(k % _NB)


def kernel(x, table):
    xr = x.reshape(_N // _K, _K)
    tp = jnp.pad(table, ((0, 0), (0, _DP - _D)))
    out = _emb_lookup(xr, tp)
    return out[:, :_D].reshape(x.shape[0], x.shape[1], _D)
